# Initial kernel scaffold; baseline (speedup 1.0000x reference)
import jax
import jax.numpy as jnp
from jax.experimental import pallas as pl


def kernel(x, edge_index, batch, W1, b1, W2, b2, W3, b3):
    raise NotImplementedError("write your pallas kernel here")


# speedup vs baseline: 8.0653x; 8.0653x over previous
import functools

import jax
import jax.numpy as jnp
from jax import lax
from jax.experimental import pallas as pl
from jax.experimental.pallas import tpu as pltpu
from jax.experimental.pallas import tpu_sc as plsc

N = 10000
NPAD = 10240
D = 128
G = 64
NC, NS = 2, 16
NW = NC * NS
CHUNK = 128
CPT = 80
NSTAGE = 2
SPT = CPT // NSTAGE
EPAD = NW * CPT * CHUNK
ROWS_PT = NPAD // NS
BM = 1024

_MESH = plsc.VectorSubcoreMesh(
    core_axis_name="c", subcore_axis_name="s", num_cores=NC, num_subcores=NS)


@functools.partial(
    pl.kernel,
    out_type=jax.ShapeDtypeStruct((NW, NPAD // 128, 128), jnp.float32),
    mesh=_MESH,
    scratch_types=[
        pltpu.VMEM((CPT, CHUNK), jnp.int32),
        pltpu.VMEM((NPAD // 128, 128), jnp.float32),
    ],
    compiler_params=pltpu.CompilerParams(needs_layout_passes=False),
)
def _deg_kernel(dst_hbm, out_hbm, dstv, counts):
    cid = lax.axis_index("c")
    sid = lax.axis_index("s")
    wid = cid * NS + sid
    zero = jnp.zeros((16,), jnp.float32)

    def _z(i, _):
        for j in range(128 // 16):
            counts[i, pl.ds(j * 16, 16)] = zero
        return 0
    lax.fori_loop(0, NPAD // 128, _z, 0)

    pltpu.sync_copy(dst_hbm.at[wid], dstv)
    one = jnp.ones((16,), jnp.float32)

    def _s(i, _):
        for j in range(CHUNK // 16):
            idx = dstv[i, pl.ds(j * 16, 16)]
            row = lax.shift_right_logical(idx, 7)
            col = lax.bitwise_and(idx, 127)
            plsc.addupdate_scatter(counts, [row, col], one)
        return 0
    lax.fori_loop(0, CPT, _s, 0)

    pltpu.sync_copy(counts, out_hbm.at[wid])


def _scatter_body(src_hbm, dst_hbm, g_hbm, out_hbm,
                  srcv, dstv, buf0, buf1, acc, sem0, sem1):
    cid = lax.axis_index("c")
    sid = lax.axis_index("s")
    wid = cid * NS + sid

    def _z(i, _):
        for j in range(D // 16):
            buf0[i, pl.ds(j * 16, 16)] = jnp.zeros((16,), jnp.float32)
        return 0
    lax.fori_loop(0, CHUNK, _z, 0)
    for r in range(ROWS_PT // CHUNK):
        pltpu.sync_copy(buf0, acc.at[pl.ds(sid * ROWS_PT + r * CHUNK, CHUNK)])
    plsc.subcore_barrier()

    for s in range(NSTAGE):
        pltpu.sync_copy(src_hbm.at[wid, pl.ds(s * SPT, SPT)], srcv)
        pltpu.sync_copy(dst_hbm.at[wid, pl.ds(s * SPT, SPT)], dstv)
        pltpu.async_copy(g_hbm.at[srcv.at[0]], buf0, sem0)

        def _step(j, _):
            pltpu.async_copy(g_hbm.at[srcv.at[2 * j + 1]], buf1, sem1)
            pltpu.make_async_copy(g_hbm.at[srcv.at[2 * j]], buf0, sem0).wait()
            pltpu.sync_copy(buf0, acc.at[dstv.at[2 * j]], add=True)

            @pl.when(j < SPT // 2 - 1)
            def _():
                pltpu.async_copy(g_hbm.at[srcv.at[2 * j + 2]], buf0, sem0)

            pltpu.make_async_copy(g_hbm.at[srcv.at[2 * j + 1]], buf1, sem1).wait()
            pltpu.sync_copy(buf1, acc.at[dstv.at[2 * j + 1]], add=True)
            return 0

        lax.fori_loop(0, SPT // 2, _step, 0)
    plsc.subcore_barrier()
    pltpu.sync_copy(acc.at[pl.ds(sid * ROWS_PT, ROWS_PT)],
                    out_hbm.at[cid, pl.ds(sid * ROWS_PT, ROWS_PT)])


@functools.partial(
    pl.kernel,
    out_type=jax.ShapeDtypeStruct((NC, NPAD, D), jnp.float32),
    mesh=_MESH,
    scratch_types=[
        pltpu.VMEM((SPT, CHUNK), jnp.int32),
        pltpu.VMEM((SPT, CHUNK), jnp.int32),
        pltpu.VMEM((CHUNK, D), jnp.float32),
        pltpu.VMEM((CHUNK, D), jnp.float32),
        pltpu.VMEM_SHARED((NPAD, D), jnp.float32),
        pltpu.SemaphoreType.DMA,
        pltpu.SemaphoreType.DMA,
    ],
)
def _scatter_kernel(src_hbm, dst_hbm, g_hbm, out_hbm,
                    srcv, dstv, buf0, buf1, acc, sem0, sem1):
    _scatter_body(src_hbm, dst_hbm, g_hbm, out_hbm,
                  srcv, dstv, buf0, buf1, acc, sem0, sem1)


def _tc_a(x_ref, w_ref, d_ref, g_ref, dinv_ref):
    deg = jnp.sum(d_ref[:, :], axis=1, keepdims=True) + 1.0
    dinv = lax.rsqrt(deg)
    h = jnp.dot(x_ref[:, :], w_ref[:, :], preferred_element_type=jnp.float32)
    g_ref[:, :] = h * dinv
    dinv_ref[:, :] = jnp.broadcast_to(dinv, (BM, D))


def _tc_a_call(xp, W1, degT):
    return pl.pallas_call(
        _tc_a,
        grid=(NPAD // BM,),
        in_specs=[
            pl.BlockSpec((BM, D), lambda i: (i, 0)),
            pl.BlockSpec((D, D), lambda i: (0, 0)),
            pl.BlockSpec((BM, NW), lambda i: (i, 0)),
        ],
        out_specs=[pl.BlockSpec((BM, D), lambda i: (i, 0)),
                   pl.BlockSpec((BM, D), lambda i: (i, 0))],
        out_shape=[jax.ShapeDtypeStruct((NPAD, D), jnp.float32),
                   jax.ShapeDtypeStruct((NPAD, D), jnp.float32)],
    )(xp, W1, degT)


def _tc_b(p0_ref, p1_ref, g_ref, dinv_ref, b_ref, w_ref, gn_ref):
    z = (p0_ref[:, :] + p1_ref[:, :] + g_ref[:, :]) * dinv_ref[:, :] + b_ref[:, :]
    z = jnp.maximum(z, 0.0)
    gn_ref[:, :] = jnp.dot(z, w_ref[:, :],
                           preferred_element_type=jnp.float32) * dinv_ref[:, :]


def _tc_b_call(p0, p1, g, dinvb, b, Wn):
    return pl.pallas_call(
        _tc_b,
        grid=(NPAD // BM,),
        in_specs=[
            pl.BlockSpec((BM, D), lambda i: (i, 0)),
            pl.BlockSpec((BM, D), lambda i: (i, 0)),
            pl.BlockSpec((BM, D), lambda i: (i, 0)),
            pl.BlockSpec((BM, D), lambda i: (i, 0)),
            pl.BlockSpec((1, D), lambda i: (0, 0)),
            pl.BlockSpec((D, D), lambda i: (0, 0)),
        ],
        out_specs=pl.BlockSpec((BM, D), lambda i: (i, 0)),
        out_shape=jax.ShapeDtypeStruct((NPAD, D), jnp.float32),
    )(p0, p1, g, dinvb, b, Wn)


def _tc_c(p0_ref, p1_ref, g_ref, dinv_ref, b_ref, bat_ref, out_ref):
    i = pl.program_id(0)
    z = (p0_ref[:, :] + p1_ref[:, :] + g_ref[:, :]) * dinv_ref[:, :] + b_ref[:, :]
    ids = bat_ref[:, :]
    col = lax.broadcasted_iota(jnp.int32, (BM, G), 1)
    onehot = (ids == col).astype(jnp.float32)
    part = lax.dot_general(onehot, z, (((0,), (0,)), ((), ())),
                           preferred_element_type=jnp.float32)

    @pl.when(i == 0)
    def _():
        out_ref[:, :] = jnp.zeros((G, D), jnp.float32)

    out_ref[:, :] += part


def _tc_c_call(p0, p1, g, dinvb, b, bat):
    return pl.pallas_call(
        _tc_c,
        grid=(NPAD // BM,),
        in_specs=[
            pl.BlockSpec((BM, D), lambda i: (i, 0)),
            pl.BlockSpec((BM, D), lambda i: (i, 0)),
            pl.BlockSpec((BM, D), lambda i: (i, 0)),
            pl.BlockSpec((BM, D), lambda i: (i, 0)),
            pl.BlockSpec((1, D), lambda i: (0, 0)),
            pl.BlockSpec((BM, 1), lambda i: (i, 0)),
        ],
        out_specs=pl.BlockSpec((G, D), lambda i: (0, 0)),
        out_shape=jax.ShapeDtypeStruct((G, D), jnp.float32),
    )(p0, p1, g, dinvb, b, bat)


def kernel(x, edge_index, batch, W1, b1, W2, b2, W3, b3):
    x = x.astype(jnp.float32)
    src = edge_index[0].astype(jnp.int32)
    dst = edge_index[1].astype(jnp.int32)
    E = src.shape[0]
    pad = EPAD - E
    srcp = jnp.concatenate([src, jnp.full((pad,), N, jnp.int32)]).reshape(NW, CPT, CHUNK)
    dstp = jnp.concatenate([dst, jnp.full((pad,), N, jnp.int32)]).reshape(NW, CPT, CHUNK)
    xp = jnp.pad(x, ((0, NPAD - N), (0, 0)))
    batp = jnp.concatenate(
        [batch.astype(jnp.int32), jnp.full((NPAD - N,), G, jnp.int32)]).reshape(NPAD, 1)

    degp = _deg_kernel(dstp)
    degT = degp.reshape(NW, NPAD).T
    g1, dinvb = _tc_a_call(xp, W1, degT)
    p1 = _scatter_kernel(srcp, dstp, g1)
    g2 = _tc_b_call(p1[0], p1[1], g1, dinvb, b1.reshape(1, D), W2)
    p2 = _scatter_kernel(srcp, dstp, g2)
    g3 = _tc_b_call(p2[0], p2[1], g2, dinvb, b2.reshape(1, D), W3)
    p3 = _scatter_kernel(srcp, dstp, g3)
    return _tc_c_call(p3[0], p3[1], g3, dinvb, b3.reshape(1, D), batp)


# speedup vs baseline: 27.8312x; 3.4507x over previous
import functools

import jax
import jax.numpy as jnp
from jax import lax
from jax.experimental import pallas as pl
from jax.experimental.pallas import tpu as pltpu
from jax.experimental.pallas import tpu_sc as plsc

N = 10000
NPAD = 10240
D = 128
G = 64
NC, NS = 2, 16
NW = NC * NS
CHUNK = 128
CPT = 80
NSTAGE = 2
SPT = CPT // NSTAGE
EPAD = NW * CPT * CHUNK
ROWS_PT = NPAD // NS
BM = 1024

_MESH = plsc.VectorSubcoreMesh(
    core_axis_name="c", subcore_axis_name="s", num_cores=NC, num_subcores=NS)


@functools.partial(
    pl.kernel,
    out_type=jax.ShapeDtypeStruct((NW, NPAD // 128, 128), jnp.float32),
    mesh=_MESH,
    scratch_types=[
        pltpu.VMEM((CPT, CHUNK), jnp.int32),
        pltpu.VMEM((NPAD // 128, 128), jnp.float32),
    ],
    compiler_params=pltpu.CompilerParams(needs_layout_passes=False),
)
def _deg_kernel(dst_hbm, out_hbm, dstv, counts):
    cid = lax.axis_index("c")
    sid = lax.axis_index("s")
    wid = cid * NS + sid
    zero = jnp.zeros((16,), jnp.float32)

    def _z(i, _):
        for j in range(128 // 16):
            counts[i, pl.ds(j * 16, 16)] = zero
        return 0
    lax.fori_loop(0, NPAD // 128, _z, 0)

    pltpu.sync_copy(dst_hbm.at[wid], dstv)
    one = jnp.ones((16,), jnp.float32)

    def _s(i, _):
        for j in range(CHUNK // 16):
            idx = dstv[i, pl.ds(j * 16, 16)]
            row = lax.shift_right_logical(idx, 7)
            col = lax.bitwise_and(idx, 127)
            plsc.addupdate_scatter(counts, [row, col], one)
        return 0
    lax.fori_loop(0, CPT, _s, 0)

    pltpu.sync_copy(counts, out_hbm.at[wid])


def _scatter_body(src_hbm, dst_hbm, g_hbm, out_hbm,
                  srcv, dstv, buf0, buf1, acc, sem0, sem1):
    cid = lax.axis_index("c")
    sid = lax.axis_index("s")
    wid = cid * NS + sid

    def _z(i, _):
        for j in range(D // 16):
            buf0[i, pl.ds(j * 16, 16)] = jnp.zeros((16,), jnp.float32)
        return 0
    lax.fori_loop(0, CHUNK, _z, 0)
    for r in range(ROWS_PT // CHUNK):
        pltpu.sync_copy(buf0, acc.at[pl.ds(sid * ROWS_PT + r * CHUNK, CHUNK)])
    plsc.subcore_barrier()

    for s in range(NSTAGE):
        pltpu.sync_copy(src_hbm.at[wid, pl.ds(s * SPT, SPT)], srcv)
        pltpu.sync_copy(dst_hbm.at[wid, pl.ds(s * SPT, SPT)], dstv)
        pltpu.async_copy(g_hbm.at[srcv.at[0]], buf0, sem0)

        def _step(j, _):
            pltpu.async_copy(g_hbm.at[srcv.at[2 * j + 1]], buf1, sem1)
            pltpu.make_async_copy(g_hbm.at[srcv.at[2 * j]], buf0, sem0).wait()
            pltpu.sync_copy(buf0, acc.at[dstv.at[2 * j]], add=True)

            @pl.when(j < SPT // 2 - 1)
            def _():
                pltpu.async_copy(g_hbm.at[srcv.at[2 * j + 2]], buf0, sem0)

            pltpu.make_async_copy(g_hbm.at[srcv.at[2 * j + 1]], buf1, sem1).wait()
            pltpu.sync_copy(buf1, acc.at[dstv.at[2 * j + 1]], add=True)
            return 0

        lax.fori_loop(0, SPT // 2, _step, 0)
    plsc.subcore_barrier()
    pltpu.sync_copy(acc.at[pl.ds(sid * ROWS_PT, ROWS_PT)],
                    out_hbm.at[cid, pl.ds(sid * ROWS_PT, ROWS_PT)])


@functools.partial(
    pl.kernel,
    out_type=jax.ShapeDtypeStruct((NC, NPAD, D), jnp.float32),
    mesh=_MESH,
    scratch_types=[
        pltpu.VMEM((SPT, CHUNK), jnp.int32),
        pltpu.VMEM((SPT, CHUNK), jnp.int32),
        pltpu.VMEM((CHUNK, D), jnp.float32),
        pltpu.VMEM((CHUNK, D), jnp.float32),
        pltpu.VMEM_SHARED((NPAD, D), jnp.float32),
        pltpu.SemaphoreType.DMA,
        pltpu.SemaphoreType.DMA,
    ],
)
def _scatter_kernel(src_hbm, dst_hbm, g_hbm, out_hbm,
                    srcv, dstv, buf0, buf1, acc, sem0, sem1):
    _scatter_body(src_hbm, dst_hbm, g_hbm, out_hbm,
                  srcv, dstv, buf0, buf1, acc, sem0, sem1)


def _tc_a(x_ref, w_ref, d_ref, g_ref, dinv_ref):
    deg = jnp.sum(d_ref[:, :], axis=1, keepdims=True) + 1.0
    dinv = lax.rsqrt(deg)
    h = jnp.dot(x_ref[:, :], w_ref[:, :], preferred_element_type=jnp.float32)
    g_ref[:, :] = h * dinv
    dinv_ref[:, :] = jnp.broadcast_to(dinv, (BM, D))


def _tc_a_call(xp, W1, degT):
    return pl.pallas_call(
        _tc_a,
        grid=(NPAD // BM,),
        in_specs=[
            pl.BlockSpec((BM, D), lambda i: (i, 0)),
            pl.BlockSpec((D, D), lambda i: (0, 0)),
            pl.BlockSpec((BM, NW), lambda i: (i, 0)),
        ],
        out_specs=[pl.BlockSpec((BM, D), lambda i: (i, 0)),
                   pl.BlockSpec((BM, D), lambda i: (i, 0))],
        out_shape=[jax.ShapeDtypeStruct((NPAD, D), jnp.float32),
                   jax.ShapeDtypeStruct((NPAD, D), jnp.float32)],
    )(xp, W1, degT)


def _tc_b(p0_ref, p1_ref, g_ref, dinv_ref, b_ref, w_ref, gn_ref):
    z = (p0_ref[:, :] + p1_ref[:, :] + g_ref[:, :]) * dinv_ref[:, :] + b_ref[:, :]
    z = jnp.maximum(z, 0.0)
    gn_ref[:, :] = jnp.dot(z, w_ref[:, :],
                           preferred_element_type=jnp.float32) * dinv_ref[:, :]


def _tc_b_call(p0, p1, g, dinvb, b, Wn):
    return pl.pallas_call(
        _tc_b,
        grid=(NPAD // BM,),
        in_specs=[
            pl.BlockSpec((BM, D), lambda i: (i, 0)),
            pl.BlockSpec((BM, D), lambda i: (i, 0)),
            pl.BlockSpec((BM, D), lambda i: (i, 0)),
            pl.BlockSpec((BM, D), lambda i: (i, 0)),
            pl.BlockSpec((1, D), lambda i: (0, 0)),
            pl.BlockSpec((D, D), lambda i: (0, 0)),
        ],
        out_specs=pl.BlockSpec((BM, D), lambda i: (i, 0)),
        out_shape=jax.ShapeDtypeStruct((NPAD, D), jnp.float32),
    )(p0, p1, g, dinvb, b, Wn)


def _tc_c(p0_ref, p1_ref, g_ref, dinv_ref, b_ref, bat_ref, out_ref):
    i = pl.program_id(0)
    z = (p0_ref[:, :] + p1_ref[:, :] + g_ref[:, :]) * dinv_ref[:, :] + b_ref[:, :]
    ids = bat_ref[:, :]
    col = lax.broadcasted_iota(jnp.int32, (BM, G), 1)
    onehot = (ids == col).astype(jnp.float32)
    part = lax.dot_general(onehot, z, (((0,), (0,)), ((), ())),
                           preferred_element_type=jnp.float32)

    @pl.when(i == 0)
    def _():
        out_ref[:, :] = jnp.zeros((G, D), jnp.float32)

    out_ref[:, :] += part


def _tc_c_call(p0, p1, g, dinvb, b, bat):
    return pl.pallas_call(
        _tc_c,
        grid=(NPAD // BM,),
        in_specs=[
            pl.BlockSpec((BM, D), lambda i: (i, 0)),
            pl.BlockSpec((BM, D), lambda i: (i, 0)),
            pl.BlockSpec((BM, D), lambda i: (i, 0)),
            pl.BlockSpec((BM, D), lambda i: (i, 0)),
            pl.BlockSpec((1, D), lambda i: (0, 0)),
            pl.BlockSpec((BM, 1), lambda i: (i, 0)),
        ],
        out_specs=pl.BlockSpec((G, D), lambda i: (0, 0)),
        out_shape=jax.ShapeDtypeStruct((G, D), jnp.float32),
    )(p0, p1, g, dinvb, b, bat)


def kernel(x, edge_index, batch, W1, b1, W2, b2, W3, b3):
    x = x.astype(jnp.float32)
    src = edge_index[0].astype(jnp.int32)
    dst = edge_index[1].astype(jnp.int32)
    E = src.shape[0]
    pad = EPAD - E
    padidx = (jnp.arange(pad, dtype=jnp.int32) % (NPAD - N)) + N
    srcp = jnp.concatenate([src, padidx]).reshape(NW, CPT, CHUNK)
    dstp = jnp.concatenate([dst, padidx]).reshape(NW, CPT, CHUNK)
    xp = jnp.pad(x, ((0, NPAD - N), (0, 0)))
    batp = jnp.concatenate(
        [batch.astype(jnp.int32), jnp.full((NPAD - N,), G, jnp.int32)]).reshape(NPAD, 1)

    degp = _deg_kernel(dstp)
    degT = degp.reshape(NW, NPAD).T
    g1, dinvb = _tc_a_call(xp, W1, degT)
    p1 = _scatter_kernel(srcp, dstp, g1)
    g2 = _tc_b_call(p1[0], p1[1], g1, dinvb, b1.reshape(1, D), W2)
    p2 = _scatter_kernel(srcp, dstp, g2)
    g3 = _tc_b_call(p2[0], p2[1], g2, dinvb, b2.reshape(1, D), W3)
    p3 = _scatter_kernel(srcp, dstp, g3)
    return _tc_c_call(p3[0], p3[1], g3, dinvb, b3.reshape(1, D), batp)
